# baseline (speedup 1.0000x reference)
import functools

import jax
import jax.numpy as jnp
from jax import lax
from jax.experimental import pallas as pl
from jax.experimental.pallas import tpu as pltpu
from jax.experimental.pallas import tpu_sc as plsc

B, S, D = 4096, 200, 64
NC, NS, L = 2, 16, 16
NW = NC * NS
BPW = B // NW
R = 12
SLOTS = R + 1

_mesh = plsc.VectorSubcoreMesh(core_axis_name="c", subcore_axis_name="s")


@functools.partial(
    pl.kernel,
    mesh=_mesh,
    out_type=jax.ShapeDtypeStruct((D, B), jnp.float32),
    scratch_types=[
        pltpu.VMEM((BPW,), jnp.int32),
        pltpu.VMEM((SLOTS, D, BPW), jnp.float32),
        pltpu.VMEM((D, BPW), jnp.float32),
        pltpu.SMEM((BPW,), jnp.int32),
        pltpu.SMEM((S,), jnp.int32),
        pltpu.SMEM((BPW,), jnp.int32),
        pltpu.SMEM((BPW,), jnp.int32),
        pltpu.SMEM((BPW,), jnp.int32),
        pltpu.SemaphoreType.DMA((SLOTS,)),
    ],
    compiler_params=pltpu.CompilerParams(needs_layout_passes=False),
)
def _fetch(seq_t_hbm, idx_hbm, out_hbm, idx_v, stage_v, cols_v,
           sidx, pres, fs, heads, nxt, sems):
    wid = lax.axis_index("s") * NC + lax.axis_index("c")
    base = wid * BPW
    pltpu.sync_copy(idx_hbm.at[pl.ds(base, BPW)], idx_v)

    lane = lax.iota(jnp.int32, L)

    def fire(k):
        slot = lax.rem(k, SLOTS)
        pltpu.async_copy(
            seq_t_hbm.at[fs[k], :, pl.ds(base, BPW)],
            stage_v.at[slot],
            sems.at[slot],
        )
        return 0

    def initp(t, carry):
        pres[t] = -1
        return carry

    lax.fori_loop(0, S, initp, 0)

    def claim(s, i, nf):
        pres[s] = nf
        fs[nf] = s
        heads[nf] = i
        nxt[i] = -1
        lax.cond(nf < R, lambda: fire(nf), lambda: 0)
        return nf + 1

    def chain(p, i, nf):
        nxt[i] = heads[p]
        heads[p] = i
        return nf

    def dedup(i, nf):
        s = sidx[i]
        p = pres[s]
        return lax.cond(p < 0, lambda: claim(s, i, nf), lambda: chain(p, i, nf))

    nf = 0
    for c in range(BPW // L):
        vobj = idx_v[pl.ds(c * L, L)]
        for k in range(L):
            sidx[c * L + k] = jnp.max(jnp.where(lane == k, vobj, 0))
        nf = lax.fori_loop(c * L, (c + 1) * L, dedup, nf)


    rows = [q * L + lane for q in range(D // L)]

    def extract(slot, i):
        col = jnp.full((L,), i, jnp.int32)
        for q in range(D // L):
            vals = plsc.load_gather(stage_v.at[slot], [rows[q], col])
            plsc.store_scatter(cols_v, [rows[q], col], vals)

    def process(k, carry):
        slot = lax.rem(k, SLOTS)
        pltpu.make_async_copy(
            seq_t_hbm.at[0, :, pl.ds(base, BPW)],
            stage_v.at[slot],
            sems.at[slot],
        ).wait()

        lax.cond(k + R < nf, lambda: fire(k + R), lambda: 0)

        def body(i):
            extract(slot, i)
            return nxt[i]

        lax.while_loop(lambda i: i >= 0, body, heads[k])
        return carry

    lax.fori_loop(0, nf, process, 0)

    pltpu.sync_copy(cols_v, out_hbm.at[:, pl.ds(base, BPW)])


def kernel(seq, obj_idx):
    seq_t = jnp.transpose(seq, (1, 2, 0))
    idx = obj_idx.astype(jnp.int32)
    out_t = _fetch(seq_t, idx)
    return jnp.transpose(out_t, (1, 0))


# speedup vs baseline: 1.0074x; 1.0074x over previous
import functools

import jax
import jax.numpy as jnp
from jax import lax
from jax.experimental import pallas as pl
from jax.experimental.pallas import tpu as pltpu
from jax.experimental.pallas import tpu_sc as plsc

B, S, D = 4096, 200, 64
NC, NS, L = 2, 16, 16
NW = NC * NS
BPW = B // NW
R = 8
SLOTS = R + 1

_mesh = plsc.VectorSubcoreMesh(core_axis_name="c", subcore_axis_name="s")


@functools.partial(
    pl.kernel,
    mesh=_mesh,
    out_type=jax.ShapeDtypeStruct((D, B), jnp.float32),
    scratch_types=[
        pltpu.VMEM((BPW,), jnp.int32),
        pltpu.VMEM((SLOTS, D, BPW), jnp.float32),
        pltpu.VMEM((D, BPW), jnp.float32),
        pltpu.SMEM((BPW,), jnp.int32),
        pltpu.SMEM((S,), jnp.int32),
        pltpu.SMEM((BPW,), jnp.int32),
        pltpu.SMEM((BPW,), jnp.int32),
        pltpu.SMEM((BPW,), jnp.int32),
        pltpu.SemaphoreType.DMA((SLOTS,)),
    ],
    compiler_params=pltpu.CompilerParams(needs_layout_passes=False),
)
def _fetch(seq_t_hbm, idx_hbm, out_hbm, idx_v, stage_v, cols_v,
           sidx, pres, fs, heads, nxt, sems):
    wid = lax.axis_index("s") * NC + lax.axis_index("c")
    base = wid * BPW
    pltpu.sync_copy(idx_hbm.at[pl.ds(base, BPW)], idx_v)

    lane = lax.iota(jnp.int32, L)

    def fire(k):
        slot = lax.rem(k, SLOTS)
        pltpu.async_copy(
            seq_t_hbm.at[fs[k], :, pl.ds(base, BPW)],
            stage_v.at[slot],
            sems.at[slot],
        )
        return 0

    def initp(t, carry):
        pres[t] = -1
        return carry

    lax.fori_loop(0, S, initp, 0)

    def claim(s, i, nf):
        pres[s] = nf
        fs[nf] = s
        heads[nf] = i
        nxt[i] = -1
        lax.cond(nf < R, lambda: fire(nf), lambda: 0)
        return nf + 1

    def chain(p, i, nf):
        nxt[i] = heads[p]
        heads[p] = i
        return nf

    def dedup(i, nf):
        s = sidx[i]
        p = pres[s]
        return lax.cond(p < 0, lambda: claim(s, i, nf), lambda: chain(p, i, nf))

    nf = 0
    for c in range(BPW // L):
        vobj = idx_v[pl.ds(c * L, L)]
        for k in range(L):
            sidx[c * L + k] = jnp.max(jnp.where(lane == k, vobj, 0))
        nf = lax.fori_loop(c * L, (c + 1) * L, dedup, nf)


    rows = [q * L + lane for q in range(D // L)]

    def extract(slot, i):
        col = jnp.full((L,), i, jnp.int32)
        for q in range(D // L):
            vals = plsc.load_gather(stage_v.at[slot], [rows[q], col])
            plsc.store_scatter(cols_v, [rows[q], col], vals)

    def process(k, carry):
        slot = lax.rem(k, SLOTS)
        pltpu.make_async_copy(
            seq_t_hbm.at[0, :, pl.ds(base, BPW)],
            stage_v.at[slot],
            sems.at[slot],
        ).wait()

        lax.cond(k + R < nf, lambda: fire(k + R), lambda: 0)

        def body(i):
            extract(slot, i)
            return nxt[i]

        lax.while_loop(lambda i: i >= 0, body, heads[k])
        return carry

    lax.fori_loop(0, nf, process, 0)

    pltpu.sync_copy(cols_v, out_hbm.at[:, pl.ds(base, BPW)])


def kernel(seq, obj_idx):
    seq_t = jnp.transpose(seq, (1, 2, 0))
    idx = obj_idx.astype(jnp.int32)
    out_t = _fetch(seq_t, idx)
    return jnp.transpose(out_t, (1, 0))


# speedup vs baseline: 1.0085x; 1.0011x over previous
import functools

import jax
import jax.numpy as jnp
from jax import lax
from jax.experimental import pallas as pl
from jax.experimental.pallas import tpu as pltpu
from jax.experimental.pallas import tpu_sc as plsc

B, S, D = 4096, 200, 64
NC, NS, L = 2, 16, 16
NW = NC * NS
BPW = B // NW
R = 8
SLOTS = R + 1

_mesh = plsc.VectorSubcoreMesh(core_axis_name="c", subcore_axis_name="s")


@functools.partial(
    pl.kernel,
    mesh=_mesh,
    out_type=jax.ShapeDtypeStruct((D, B), jnp.float32),
    scratch_types=[
        pltpu.VMEM((BPW,), jnp.int32),
        pltpu.VMEM((SLOTS, D, BPW), jnp.float32),
        pltpu.VMEM((D, BPW), jnp.float32),
        pltpu.SMEM((BPW,), jnp.int32),
        pltpu.SMEM((S,), jnp.int32),
        pltpu.SMEM((BPW,), jnp.int32),
        pltpu.SMEM((BPW,), jnp.int32),
        pltpu.SMEM((BPW,), jnp.int32),
        pltpu.SemaphoreType.DMA((SLOTS,)),
    ],
    compiler_params=pltpu.CompilerParams(
        needs_layout_passes=False, disable_bounds_checks=True
    ),
)
def _fetch(seq_t_hbm, idx_hbm, out_hbm, idx_v, stage_v, cols_v,
           sidx, pres, fs, heads, nxt, sems):
    wid = lax.axis_index("s") * NC + lax.axis_index("c")
    base = wid * BPW
    pltpu.sync_copy(idx_hbm.at[pl.ds(base, BPW)], idx_v)

    lane = lax.iota(jnp.int32, L)

    def fire(k):
        slot = lax.rem(k, SLOTS)
        pltpu.async_copy(
            seq_t_hbm.at[fs[k], :, pl.ds(base, BPW)],
            stage_v.at[slot],
            sems.at[slot],
        )
        return 0

    def initp(t, carry):
        pres[t] = -1
        return carry

    lax.fori_loop(0, S, initp, 0)

    def claim(s, i, nf):
        pres[s] = nf
        fs[nf] = s
        heads[nf] = i
        nxt[i] = -1
        lax.cond(nf < R, lambda: fire(nf), lambda: 0)
        return nf + 1

    def chain(p, i, nf):
        nxt[i] = heads[p]
        heads[p] = i
        return nf

    def dedup(i, nf):
        s = sidx[i]
        p = pres[s]
        return lax.cond(p < 0, lambda: claim(s, i, nf), lambda: chain(p, i, nf))

    nf = 0
    for c in range(BPW // L):
        vobj = idx_v[pl.ds(c * L, L)]
        for k in range(L):
            sidx[c * L + k] = jnp.max(jnp.where(lane == k, vobj, 0))
        nf = lax.fori_loop(c * L, (c + 1) * L, dedup, nf)


    rows = [q * L + lane for q in range(D // L)]

    def extract(slot, i):
        col = jnp.full((L,), i, jnp.int32)
        for q in range(D // L):
            vals = plsc.load_gather(stage_v.at[slot], [rows[q], col])
            plsc.store_scatter(cols_v, [rows[q], col], vals)

    def process(k, carry):
        slot = lax.rem(k, SLOTS)
        pltpu.make_async_copy(
            seq_t_hbm.at[0, :, pl.ds(base, BPW)],
            stage_v.at[slot],
            sems.at[slot],
        ).wait()

        lax.cond(k + R < nf, lambda: fire(k + R), lambda: 0)

        def body(i):
            extract(slot, i)
            return nxt[i]

        lax.while_loop(lambda i: i >= 0, body, heads[k])
        return carry

    lax.fori_loop(0, nf, process, 0)

    pltpu.sync_copy(cols_v, out_hbm.at[:, pl.ds(base, BPW)])


def kernel(seq, obj_idx):
    seq_t = jnp.transpose(seq, (1, 2, 0))
    idx = obj_idx.astype(jnp.int32)
    out_t = _fetch(seq_t, idx)
    return jnp.transpose(out_t, (1, 0))


# speedup vs baseline: 1.0100x; 1.0015x over previous
import functools

import jax
import jax.numpy as jnp
from jax import lax
from jax.experimental import pallas as pl
from jax.experimental.pallas import tpu as pltpu
from jax.experimental.pallas import tpu_sc as plsc

B, S, D = 4096, 200, 64
NC, NS, L = 2, 16, 16
NW = NC * NS
BPW = B // NW
R = 8
SLOTS = R + 1

_mesh = plsc.VectorSubcoreMesh(core_axis_name="c", subcore_axis_name="s")


@functools.partial(
    pl.kernel,
    mesh=_mesh,
    out_type=jax.ShapeDtypeStruct((D, B), jnp.float32),
    scratch_types=[
        pltpu.VMEM((BPW,), jnp.int32),
        pltpu.VMEM((SLOTS, D, BPW), jnp.float32),
        pltpu.VMEM((D, BPW), jnp.float32),
        pltpu.SMEM((BPW,), jnp.int32),
        pltpu.SMEM((S,), jnp.int32),
        pltpu.SMEM((BPW,), jnp.int32),
        pltpu.SMEM((BPW,), jnp.int32),
        pltpu.SMEM((BPW,), jnp.int32),
        pltpu.SemaphoreType.DMA((SLOTS,)),
    ],
    compiler_params=pltpu.CompilerParams(needs_layout_passes=False),
)
def _fetch(seq_t_hbm, idx_hbm, out_hbm, idx_v, stage_v, cols_v,
           sidx, pres, fs, heads, nxt, sems):
    wid = lax.axis_index("s") * NC + lax.axis_index("c")
    base = wid * BPW
    pltpu.sync_copy(idx_hbm.at[pl.ds(base, BPW)], idx_v)

    lane = lax.iota(jnp.int32, L)

    def fire(k):
        slot = lax.rem(k, SLOTS)
        pltpu.async_copy(
            seq_t_hbm.at[fs[k], :, pl.ds(base, BPW)],
            stage_v.at[slot],
            sems.at[slot],
        )
        return 0

    def initp(t, carry):
        pres[t] = -1
        return carry

    lax.fori_loop(0, S, initp, 0)

    def claim(s, i, nf):
        pres[s] = nf
        fs[nf] = s
        heads[nf] = i
        nxt[i] = -1
        lax.cond(nf < R, lambda: fire(nf), lambda: 0)
        return nf + 1

    def chain(p, i, nf):
        nxt[i] = heads[p]
        heads[p] = i
        return nf

    def dedup(i, nf):
        s = sidx[i]
        p = pres[s]
        return lax.cond(p < 0, lambda: claim(s, i, nf), lambda: chain(p, i, nf))

    nf = 0
    for c in range(BPW // L):
        vobj = idx_v[pl.ds(c * L, L)]
        for k in range(L):
            sidx[c * L + k] = jnp.max(jnp.where(lane == k, vobj, 0))
        nf = lax.fori_loop(c * L, (c + 1) * L, dedup, nf)


    rows = [q * L + lane for q in range(D // L)]

    def extract(slot, i):
        col = jnp.full((L,), i, jnp.int32)
        for q in range(D // L):
            vals = plsc.load_gather(stage_v.at[slot], [rows[q], col])
            plsc.store_scatter(cols_v, [rows[q], col], vals)

    def process(k, carry):
        slot = lax.rem(k, SLOTS)
        pltpu.make_async_copy(
            seq_t_hbm.at[0, :, pl.ds(base, BPW)],
            stage_v.at[slot],
            sems.at[slot],
        ).wait()

        lax.cond(k + R < nf, lambda: fire(k + R), lambda: 0)

        def body(i):
            extract(slot, i)
            return nxt[i]

        lax.while_loop(lambda i: i >= 0, body, heads[k])
        return carry

    lax.fori_loop(0, nf, process, 0)

    pltpu.sync_copy(cols_v, out_hbm.at[:, pl.ds(base, BPW)])


def kernel(seq, obj_idx):
    seq_t = jnp.transpose(seq, (1, 2, 0))
    idx = obj_idx.astype(jnp.int32)
    out_t = _fetch(seq_t, idx)
    return jnp.transpose(out_t, (1, 0))
